# baseline (speedup 1.0000x reference)
import jax
import jax.numpy as jnp
from jax import lax
from jax.experimental import pallas as pl
from jax.experimental.pallas import tpu as pltpu
from jax.experimental.pallas import tpu_sc as plsc

_EMBED = 32
_NW = 32
_TPW = 200
_BT = 10
_NB = _TPW // _BT


def _emb_body(idx_hbm, table_hbm, out_hbm, idx_v, g_v, t_v, sem_g, sem_o):
    n = idx_hbm.shape[0]
    per_w = n // _NW
    wid = lax.axis_index("s") * 2 + lax.axis_index("c")
    base = wid * per_w

    pltpu.sync_copy(idx_hbm.at[pl.ds(base, per_w)], idx_v)

    br = _BT * 128

    def gather(k, b):
        return pltpu.make_async_copy(
            table_hbm.at[idx_v.at[pl.ds(k * br, br)]], g_v.at[b], sem_g.at[b])

    def wb_copy(beta, te, dst_row):
        return pltpu.make_async_copy(
            t_v.at[beta, pl.ds(te * 8, 8)],
            out_hbm.at[pl.ds(dst_row, 8)], sem_o.at[beta])

    base_iota = lax.iota(jnp.int32, 16)
    cols = [jnp.full((16,), e, jnp.int32) for e in range(_EMBED)]

    gather(0, 0).start()
    gather(1, 1).start()

    def batch(k, carry):
        b = lax.rem(k, 2)
        gather(k, b).wait()

        def sub_tile(u, carry2):
            beta = lax.rem(u, 4)

            @pl.when(k * _BT + u >= 4)
            def _():
                for _ in range(4):
                    wb_copy(beta, 0, 0).wait()

            @plsc.parallel_loop(0, 8 * _EMBED, unroll=16)
            def _(q):
                jg = lax.shift_right_logical(q, 5)
                e = lax.bitwise_and(q, _EMBED - 1)
                rows = base_iota + (u * 128 + 16 * jg)
                colv = lax.broadcast(e, (16,))
                vals = plsc.load_gather(g_v.at[b], [rows, colv])
                off = pl.multiple_of(16 * jg, 16)
                t_v[beta, e, pl.ds(off, 16)] = vals

            tau = wid * _TPW + k * _BT + u
            h = tau // 32
            tb = tau - h * 32
            for te in range(4):
                wb_copy(beta, te, ((h * 4 + te) * 32 + tb) * 8).start()
            return carry2

        lax.fori_loop(0, _BT, sub_tile, 0)

        @pl.when(k + 2 < _NB)
        def _():
            gather(k + 2, b).start()

        return carry

    lax.fori_loop(0, _NB, batch, 0)

    for beta in range(4):
        for _ in range(4):
            wb_copy(beta, 0, 0).wait()


def kernel(x_ids, table):
    b, h = x_ids.shape
    n = b * h
    idx = x_ids.T.reshape(n).astype(jnp.int32)
    table_wide = lax.optimization_barrier(
        table.reshape(table.shape[0] * _EMBED // 128, 128))
    table_lin = table_wide.reshape(table.shape[0], _EMBED)
    run = pl.kernel(
        _emb_body,
        mesh=plsc.VectorSubcoreMesh(core_axis_name="c", subcore_axis_name="s"),
        out_type=jax.ShapeDtypeStruct((n * _EMBED // 128, 128), jnp.float32),
        scratch_types=[
            pltpu.VMEM((n // _NW,), jnp.int32),
            pltpu.VMEM((2, _BT * 128, _EMBED), jnp.float32),
            pltpu.VMEM((4, _EMBED, 128), jnp.float32),
            pltpu.SemaphoreType.DMA((2,)),
            pltpu.SemaphoreType.DMA((4,)),
        ],
        compiler_params=pltpu.CompilerParams(
            use_tc_tiling_on_sc=False, needs_layout_passes=False),
    )
    out = run(idx, table_lin)
    out5 = out.reshape(h, 4, b // 128, 8, 128)
    return out5.transpose(2, 4, 0, 1, 3).reshape(b, h, _EMBED)


# speedup vs baseline: 1.4238x; 1.4238x over previous
import jax
import jax.numpy as jnp
from jax import lax
from jax.experimental import pallas as pl
from jax.experimental.pallas import tpu as pltpu
from jax.experimental.pallas import tpu_sc as plsc

_EMBED = 32
_NW = 32
_TPW = 200
_BT = 10
_NB = _TPW // _BT


def _emb_body(idx_hbm, table_hbm, out_hbm, idx_v, g_v, t_v, sem_g, sem_o):
    n = idx_hbm.shape[0]
    per_w = n // _NW
    wid = lax.axis_index("s") * 2 + lax.axis_index("c")
    base = wid * per_w

    pltpu.sync_copy(idx_hbm.at[pl.ds(base, per_w)], idx_v)

    br = _BT * 128

    def gather(k, b):
        return pltpu.make_async_copy(
            table_hbm.at[idx_v.at[pl.ds(k * br, br)]], g_v.at[b], sem_g.at[b])

    def wb_copy(beta, te, dst_row):
        return pltpu.make_async_copy(
            t_v.at[beta, pl.ds(te * 8, 8)],
            out_hbm.at[pl.ds(dst_row, 8)], sem_o.at[beta])

    base_iota = lax.iota(jnp.int32, 16)
    cols = [jnp.full((16,), e, jnp.int32) for e in range(_EMBED)]

    gather(0, 0).start()
    gather(1, 1).start()

    def batch(k, carry):
        b = lax.rem(k, 2)
        gather(k, b).wait()

        def sub_tile(u, carry2):
            beta = lax.rem(u, 4)

            @pl.when(k * _BT + u >= 4)
            def _():
                for _ in range(4):
                    wb_copy(beta, 0, 0).wait()

            @plsc.parallel_loop(0, 16 * 16, unroll=8)
            def _(q):
                jg = lax.shift_right_logical(q, 5)
                eh = lax.bitwise_and(lax.shift_right_logical(q, 4), 1)
                d = lax.bitwise_and(q, 15)
                rows_r = base_iota + (u * 128 + 16 * jg)
                diag = lax.bitwise_and(base_iota + d, 15) + eh * 16
                cols_w = base_iota + 16 * jg
                vals = plsc.load_gather(g_v.at[b], [rows_r, diag])
                plsc.store_scatter(t_v.at[beta], [diag, cols_w], vals)

            tau = wid * _TPW + k * _BT + u
            h = tau // 32
            tb = tau - h * 32
            for te in range(4):
                wb_copy(beta, te, ((h * 4 + te) * 32 + tb) * 8).start()
            return carry2

        lax.fori_loop(0, _BT, sub_tile, 0)

        @pl.when(k + 2 < _NB)
        def _():
            gather(k + 2, b).start()

        return carry

    lax.fori_loop(0, _NB, batch, 0)

    for beta in range(4):
        for _ in range(4):
            wb_copy(beta, 0, 0).wait()


def kernel(x_ids, table):
    b, h = x_ids.shape
    n = b * h
    idx = x_ids.T.reshape(n).astype(jnp.int32)
    table_wide = lax.optimization_barrier(
        table.reshape(table.shape[0] * _EMBED // 128, 128))
    table_lin = table_wide.reshape(table.shape[0], _EMBED)
    run = pl.kernel(
        _emb_body,
        mesh=plsc.VectorSubcoreMesh(core_axis_name="c", subcore_axis_name="s"),
        out_type=jax.ShapeDtypeStruct((n * _EMBED // 128, 128), jnp.float32),
        scratch_types=[
            pltpu.VMEM((n // _NW,), jnp.int32),
            pltpu.VMEM((2, _BT * 128, _EMBED), jnp.float32),
            pltpu.VMEM((4, _EMBED, 128), jnp.float32),
            pltpu.SemaphoreType.DMA((2,)),
            pltpu.SemaphoreType.DMA((4,)),
        ],
        compiler_params=pltpu.CompilerParams(
            use_tc_tiling_on_sc=False, needs_layout_passes=False),
    )
    out = run(idx, table_lin)
    out5 = out.reshape(h, 4, b // 128, 8, 128)
    return out5.transpose(2, 4, 0, 1, 3).reshape(b, h, _EMBED)


# speedup vs baseline: 1.4529x; 1.0205x over previous
import jax
import jax.numpy as jnp
from jax import lax
from jax.experimental import pallas as pl
from jax.experimental.pallas import tpu as pltpu
from jax.experimental.pallas import tpu_sc as plsc

_EMBED = 32
_NW = 32
_TPW = 200
_BT = 10
_NB = _TPW // _BT


def _emb_body(idx_hbm, table_hbm, out_hbm, idx_v, g_v, t_v, sem_g, sem_o):
    n = idx_hbm.shape[0]
    per_w = n // _NW
    wid = lax.axis_index("s") * 2 + lax.axis_index("c")
    base = wid * per_w

    pltpu.sync_copy(idx_hbm.at[pl.ds(base, per_w)], idx_v)

    br = _BT * 128

    def gather(k, b):
        return pltpu.make_async_copy(
            table_hbm.at[idx_v.at[pl.ds(k * br, br)]], g_v.at[b], sem_g.at[b])

    def wb_copy(beta, te, dst_row):
        return pltpu.make_async_copy(
            t_v.at[beta, pl.ds(te * 8, 8)],
            out_hbm.at[pl.ds(dst_row, 8)], sem_o.at[beta])

    base_iota = lax.iota(jnp.int32, 16)
    cols = [jnp.full((16,), e, jnp.int32) for e in range(_EMBED)]

    gather(0, 0).start()
    gather(1, 1).start()

    def batch(k, carry):
        b = lax.rem(k, 2)
        gather(k, b).wait()

        def sub_tile(u, carry2):
            beta = lax.rem(u, 4)

            @pl.when(k * _BT + u >= 4)
            def _():
                for _ in range(4):
                    wb_copy(beta, 0, 0).wait()

            @plsc.parallel_loop(0, 16 * 16, unroll=16)
            def _(q):
                jg = lax.shift_right_logical(q, 5)
                eh = lax.bitwise_and(lax.shift_right_logical(q, 4), 1)
                d = lax.bitwise_and(q, 15)
                rows_r = base_iota + (u * 128 + 16 * jg)
                diag = lax.bitwise_and(base_iota + d, 15) + eh * 16
                cols_w = base_iota + 16 * jg
                vals = plsc.load_gather(g_v.at[b], [rows_r, diag])
                plsc.store_scatter(t_v.at[beta], [diag, cols_w], vals)

            tau = wid * _TPW + k * _BT + u
            h = tau // 32
            tb = tau - h * 32
            for te in range(4):
                wb_copy(beta, te, ((h * 4 + te) * 32 + tb) * 8).start()
            return carry2

        lax.fori_loop(0, _BT, sub_tile, 0)

        @pl.when(k + 2 < _NB)
        def _():
            gather(k + 2, b).start()

        return carry

    lax.fori_loop(0, _NB, batch, 0)

    for beta in range(4):
        for _ in range(4):
            wb_copy(beta, 0, 0).wait()


def kernel(x_ids, table):
    b, h = x_ids.shape
    n = b * h
    idx = x_ids.T.reshape(n).astype(jnp.int32)
    table_wide = lax.optimization_barrier(
        table.reshape(table.shape[0] * _EMBED // 128, 128))
    table_lin = table_wide.reshape(table.shape[0], _EMBED)
    run = pl.kernel(
        _emb_body,
        mesh=plsc.VectorSubcoreMesh(core_axis_name="c", subcore_axis_name="s"),
        out_type=jax.ShapeDtypeStruct((n * _EMBED // 128, 128), jnp.float32),
        scratch_types=[
            pltpu.VMEM((n // _NW,), jnp.int32),
            pltpu.VMEM((2, _BT * 128, _EMBED), jnp.float32),
            pltpu.VMEM((4, _EMBED, 128), jnp.float32),
            pltpu.SemaphoreType.DMA((2,)),
            pltpu.SemaphoreType.DMA((4,)),
        ],
        compiler_params=pltpu.CompilerParams(
            use_tc_tiling_on_sc=False, needs_layout_passes=False),
    )
    out = run(idx, table_lin)
    out5 = out.reshape(h, 4, b // 128, 8, 128)
    return out5.transpose(2, 4, 0, 1, 3).reshape(b, h, _EMBED)
